# Initial kernel scaffold; baseline (speedup 1.0000x reference)
import jax
import jax.numpy as jnp
from jax.experimental import pallas as pl


def kernel(x, edge_weight, W_fixed, b_fixed, emb_src, emb_tgt, W_adapt, b_adapt, Wf, bf, Wg, bg, Wr, br, Ws, bs, gamma, beta, edge_index):
    raise NotImplementedError("write your pallas kernel here")


# speedup vs baseline: 145.1346x; 145.1346x over previous
import jax
import jax.numpy as jnp
from jax.experimental import pallas as pl
from jax.experimental.pallas import tpu as pltpu

N = 400
CH = 64
B, T = 2, 16
G = B * T
ROWS = G * N
E = 6400
TOPK = 40
EC = 640
LN_EPS = 1e-5
F32 = jnp.float32


def _body(src_ref, dst_ref, ew_ref, es_ref, et_ref, x_ref,
          wfix_ref, wada_ref, wf0_ref, wf1_ref, wg0_ref, wg1_ref,
          wr_ref, ws_ref, bsum_ref, bf_ref, bg_ref, br_ref, bs_ref,
          gam_ref, bet_ref,
          res_ref, skip_ref, h_scr):
    f32 = F32

    lane_n = jax.lax.broadcasted_iota(jnp.int32, (1, N), 1)
    A = jnp.zeros((N, N), f32)
    for ci in range(E // EC):
        s = ci * EC
        srcs = src_ref[pl.ds(s, EC), :]
        dsts = dst_ref[pl.ds(s, EC), :]
        ews = ew_ref[pl.ds(s, EC), :]
        oh_src_w = (srcs == lane_n).astype(f32) * ews
        oh_dst = (dsts == lane_n).astype(f32)
        A = A + jax.lax.dot_general(
            oh_dst, oh_src_w, (((0,), (0,)), ((), ())),
            preferred_element_type=f32)
    deg_f = jnp.sum(A, axis=1, keepdims=True) + 1.0
    dinv_f = jax.lax.rsqrt(deg_f)

    S = jax.lax.dot_general(es_ref[...], et_ref[...],
                            (((1,), (1,)), ((), ())),
                            preferred_element_type=f32)
    S = jnp.maximum(S, 0.0)
    m = jnp.max(S, axis=1, keepdims=True)
    ex = jnp.exp(S - m)
    P = ex / jnp.sum(ex, axis=1, keepdims=True)

    lo0 = jnp.zeros((N, 1), jnp.int32)
    hi0 = jnp.full((N, 1), 0x3F800001, jnp.int32)

    def bis(_, lohi):
        lo, hi = lohi
        mid = lo + jax.lax.shift_right_logical(hi - lo, 1)
        midf = jax.lax.bitcast_convert_type(mid, f32)
        cnt = jnp.sum((P > midf).astype(f32), axis=1, keepdims=True)
        ge = cnt >= float(TOPK)
        return (jnp.where(ge, mid, lo), jnp.where(ge, hi, mid))

    lo, hi = jax.lax.fori_loop(0, 31, bis, (lo0, hi0))
    vk = jax.lax.bitcast_convert_type(hi, f32)

    gt = P > vk
    c_gt = jnp.sum(gt.astype(f32), axis=1, keepdims=True)
    eqf = (P == vk).astype(f32)
    iota_r = jax.lax.broadcasted_iota(jnp.int32, (N, N), 0)
    iota_c = jax.lax.broadcasted_iota(jnp.int32, (N, N), 1)
    lt_mat = (iota_r < iota_c).astype(f32)
    eq_before = jax.lax.dot_general(eqf, lt_mat, (((1,), (0,)), ((), ())),
                                    preferred_element_type=f32)
    mask = gt | ((P == vk) & (eq_before < (float(TOPK) - c_gt)))
    Pm = jnp.where(mask, P, 0.0)
    eye = (iota_r == iota_c).astype(f32)
    Aa = jax.lax.dot_general(Pm, eye, (((0,), (0,)), ((), ())),
                             preferred_element_type=f32)
    deg_a = jnp.sum(Aa, axis=1, keepdims=True) + 1.0
    dinv_a = jax.lax.rsqrt(deg_a)

    bsum = bsum_ref[...]
    wfix = wfix_ref[...]
    wada = wada_ref[...]

    def mp(g, _):
        s = pl.multiple_of(g * N, N)
        xs = x_ref[pl.ds(s, N), :]
        xf = jnp.dot(xs, wfix, preferred_element_type=f32) * dinv_f
        xa = jnp.dot(xs, wada, preferred_element_type=f32) * dinv_a
        hg = (dinv_f * (jnp.dot(A, xf, preferred_element_type=f32) + xf)
              + dinv_a * (jnp.dot(Aa, xa, preferred_element_type=f32) + xa)
              + bsum)
        h_scr[pl.ds(s, N), :] = hg
        return 0

    jax.lax.fori_loop(0, G, mp, 0)

    wf0 = wf0_ref[...]
    wf1 = wf1_ref[...]
    wg0 = wg0_ref[...]
    wg1 = wg1_ref[...]
    wr = wr_ref[...]
    ws = ws_ref[...]
    gam = gam_ref[...]
    bet = bet_ref[...]

    def pw(g, _):
        s = pl.multiple_of(g * N, N)
        t = jax.lax.rem(g, T)
        valid = (t >= 2).astype(f32)
        gm2 = jnp.maximum(g - 2, 0)
        s2 = pl.multiple_of(gm2 * N, N)
        h_t = h_scr[pl.ds(s, N), :]
        h_p = h_scr[pl.ds(s2, N), :] * valid
        f = (jnp.dot(h_p, wf0, preferred_element_type=f32)
             + jnp.dot(h_t, wf1, preferred_element_type=f32) + bf_ref[...])
        gg = (jnp.dot(h_p, wg0, preferred_element_type=f32)
              + jnp.dot(h_t, wg1, preferred_element_type=f32) + bg_ref[...])
        gated = jnp.tanh(f) * (1.0 / (1.0 + jnp.exp(-gg)))
        skip_ref[pl.ds(s, N), :] = (
            jnp.dot(gated, ws, preferred_element_type=f32) + bs_ref[...])
        r0 = (jnp.dot(gated, wr, preferred_element_type=f32) + br_ref[...]
              + x_ref[pl.ds(s, N), :])
        mu = jnp.mean(r0, axis=1, keepdims=True)
        var = jnp.mean((r0 - mu) ** 2, axis=1, keepdims=True)
        res_ref[pl.ds(s, N), :] = ((r0 - mu) * jax.lax.rsqrt(var + LN_EPS)
                                   * gam + bet)
        return 0

    jax.lax.fori_loop(0, G, pw, 0)


def kernel(x, edge_weight, W_fixed, b_fixed, emb_src, emb_tgt, W_adapt,
           b_adapt, Wf, bf, Wg, bg, Wr, br, Ws, bs, gamma, beta, edge_index):
    x_flat = x.reshape(ROWS, CH)
    src = edge_index[0].astype(jnp.int32).reshape(E, 1)
    dst = edge_index[1].astype(jnp.int32).reshape(E, 1)
    ew = edge_weight.reshape(E, 1)
    r2 = lambda v: v.reshape(1, CH)
    res, skip = pl.pallas_call(
        _body,
        out_shape=[jax.ShapeDtypeStruct((ROWS, CH), F32),
                   jax.ShapeDtypeStruct((ROWS, CH), F32)],
        scratch_shapes=[pltpu.VMEM((ROWS, CH), F32)],
    )(src, dst, ew, emb_src, emb_tgt, x_flat,
      W_fixed, W_adapt, Wf[:, :, 0].T, Wf[:, :, 1].T, Wg[:, :, 0].T,
      Wg[:, :, 1].T, Wr[:, :, 0].T, Ws[:, :, 0].T,
      r2(b_fixed + b_adapt), r2(bf), r2(bg), r2(br), r2(bs),
      r2(gamma), r2(beta))
    return (res.reshape(B, T, N, CH), skip.reshape(B, T, N, CH))
